# baseline (speedup 1.0000x reference)
import functools

import jax
import jax.numpy as jnp
from jax import lax
from jax.experimental import pallas as pl
from jax.experimental.pallas import tpu as pltpu
from jax.experimental.pallas import tpu_sc as plsc

B = 16384
NC = 1
NS = 8
NW = NC * NS
BPW = B // NW
NCH = 4
CH = BPW // NCH


@jax.jit
def _sc_lookup(idx, table):
    mesh = plsc.VectorSubcoreMesh(core_axis_name="c", subcore_axis_name="s",
                                  num_cores=NC, num_subcores=NS)

    @functools.partial(
        pl.kernel,
        mesh=mesh,
        out_type=jax.ShapeDtypeStruct((B,), jnp.int32),
        scratch_types=[
            pltpu.VMEM((BPW,), jnp.int32),
            pltpu.VMEM((BPW,), jnp.int32),
            pltpu.SemaphoreType.DMA,
            pltpu.SemaphoreType.DMA,
            pltpu.SemaphoreType.DMA,
            pltpu.SemaphoreType.DMA,
            pltpu.SemaphoreType.DMA,
        ],
    )
    def k(idx_hbm, table_hbm, out_hbm, idx_v, val_v, g0, g1, g2, g3, osem):
        gsem = [g0, g1, g2, g3]
        wid = lax.axis_index("s") * NC + lax.axis_index("c")
        base = wid * BPW
        pltpu.sync_copy(idx_hbm.at[pl.ds(base, BPW)], idx_v)
        gathers = [
            pltpu.async_copy(table_hbm.at[idx_v.at[pl.ds(j * CH, CH)]],
                             val_v.at[pl.ds(j * CH, CH)], gsem[j])
            for j in range(NCH)
        ]
        outs = []
        for j in range(NCH):
            gathers[j].wait()
            outs.append(
                pltpu.async_copy(val_v.at[pl.ds(j * CH, CH)],
                                 out_hbm.at[pl.ds(base + j * CH, CH)], osem))
        for c in outs:
            c.wait()

    return k(idx, table)


def kernel(idx, lookup_table):
    return _sc_lookup(idx.astype(jnp.int32), lookup_table.astype(jnp.int32))


# speedup vs baseline: 1.0565x; 1.0565x over previous
import functools

import jax
import jax.numpy as jnp
from jax import lax
from jax.experimental import pallas as pl
from jax.experimental.pallas import tpu as pltpu
from jax.experimental.pallas import tpu_sc as plsc

B = 16384
NC = 1
NS = 16
NW = NC * NS
BPW = B // NW
NCH = 4
CH = BPW // NCH


@jax.jit
def _sc_lookup(idx, table):
    mesh = plsc.VectorSubcoreMesh(core_axis_name="c", subcore_axis_name="s",
                                  num_cores=NC, num_subcores=NS)

    @functools.partial(
        pl.kernel,
        mesh=mesh,
        out_type=jax.ShapeDtypeStruct((B,), jnp.int32),
        scratch_types=[
            pltpu.VMEM((BPW,), jnp.int32),
            pltpu.VMEM((BPW,), jnp.int32),
            pltpu.SemaphoreType.DMA,
            pltpu.SemaphoreType.DMA,
            pltpu.SemaphoreType.DMA,
            pltpu.SemaphoreType.DMA,
            pltpu.SemaphoreType.DMA,
        ],
    )
    def k(idx_hbm, table_hbm, out_hbm, idx_v, val_v, g0, g1, g2, g3, osem):
        gsem = [g0, g1, g2, g3]
        wid = lax.axis_index("s") * NC + lax.axis_index("c")
        base = wid * BPW
        pltpu.sync_copy(idx_hbm.at[pl.ds(base, BPW)], idx_v)
        gathers = [
            pltpu.async_copy(table_hbm.at[idx_v.at[pl.ds(j * CH, CH)]],
                             val_v.at[pl.ds(j * CH, CH)], gsem[j])
            for j in range(NCH)
        ]
        outs = []
        for j in range(NCH):
            gathers[j].wait()
            outs.append(
                pltpu.async_copy(val_v.at[pl.ds(j * CH, CH)],
                                 out_hbm.at[pl.ds(base + j * CH, CH)], osem))
        for c in outs:
            c.wait()

    return k(idx, table)


def kernel(idx, lookup_table):
    return _sc_lookup(idx.astype(jnp.int32), lookup_table.astype(jnp.int32))
